# baseline (speedup 1.0000x reference)
import functools

import jax
import jax.numpy as jnp
from jax import lax
from jax.experimental import pallas as pl
from jax.experimental.pallas import tpu as pltpu
from jax.experimental.pallas import tpu_sc as plsc

N = 10000
E = 320000
NC = 2
NS = 16
NW = NC * NS
K = 128
CPW = 80
E_PAD = NW * CPW * K
N_ACC = 10112
RPT = N_ACC // NS
DEG_W = 8
NBUF = 4
LAG = 2


def _make_seg_pass(D, with_deg, dtype, cpw0, cpw1):
    assert cpw0 % NBUF == 0 and cpw1 % NBUF == 0
    assert min(cpw0, cpw1) >= NBUF + LAG
    cpw_max = max(cpw0, cpw1)
    out_type = [jax.ShapeDtypeStruct((NC, N_ACC, D), dtype)]
    if with_deg:
        out_type.append(jax.ShapeDtypeStruct((NC, N_ACC, DEG_W), jnp.float32))
    scratch = [
        pltpu.VMEM((cpw_max, K), jnp.int32),
        pltpu.VMEM((cpw_max, K), jnp.int32),
        pltpu.VMEM((NBUF, K, D), dtype),
        pltpu.VMEM((RPT // 4, D), dtype),
        pltpu.VMEM_SHARED((N_ACC, D), dtype),
        pltpu.SemaphoreType.DMA((NBUF,)),
        pltpu.SemaphoreType.DMA((NBUF,)),
    ]
    if with_deg:
        scratch.append(pltpu.VMEM((K, DEG_W), jnp.float32))
        scratch.append(pltpu.VMEM_SHARED((N_ACC, DEG_W), jnp.float32))
        scratch.append(pltpu.SemaphoreType.DMA)

    def body(h_hbm, srcp_hbm, dstp_hbm, *rest):
        if with_deg:
            (zeros_deg_hbm, ones_hbm, out_hbm, deg_out_hbm,
             src_v, dst_v, bufs, zbuf, acc, gsems, ssems, ones_v, dacc,
             dsem) = rest
        else:
            (out_hbm, src_v, dst_v, bufs, zbuf, acc, gsems, ssems) = rest
        c = lax.axis_index("c")
        s = lax.axis_index("s")
        count = jnp.where(c == 0, cpw0, cpw1)

        @pl.when(c == 0)
        def _():
            base = s * cpw0
            pltpu.sync_copy(srcp_hbm.at[pl.ds(base, cpw0)],
                            src_v.at[pl.ds(0, cpw0)])
            pltpu.sync_copy(dstp_hbm.at[pl.ds(base, cpw0)],
                            dst_v.at[pl.ds(0, cpw0)])

        @pl.when(c == 1)
        def _():
            base = NS * cpw0 + s * cpw1
            pltpu.sync_copy(srcp_hbm.at[pl.ds(base, cpw1)],
                            src_v.at[pl.ds(0, cpw1)])
            pltpu.sync_copy(dstp_hbm.at[pl.ds(base, cpw1)],
                            dst_v.at[pl.ds(0, cpw1)])

        lanes = 32 if dtype == jnp.bfloat16 else 16
        zvec = jnp.zeros((lanes,), dtype)

        def zrow(r, _):
            for kk in range(D // lanes):
                zbuf[r, pl.ds(kk * lanes, lanes)] = zvec
            return _

        lax.fori_loop(0, RPT // 4, zrow, None)
        rows = pl.ds(s * RPT, RPT)
        for kk in range(4):
            pltpu.sync_copy(zbuf, acc.at[pl.ds(s * RPT + kk * (RPT // 4),
                                               RPT // 4)])
        if with_deg:
            pltpu.sync_copy(zeros_deg_hbm.at[rows], dacc.at[rows])
            pltpu.sync_copy(ones_hbm, ones_v)
        plsc.subcore_barrier()

        def gather_start(j, b):
            pltpu.async_copy(h_hbm.at[src_v.at[j]], bufs.at[b], gsems.at[b])

        for b in range(NBUF):
            gather_start(b, b)

        def step(i, _):
            for b in range(NBUF):
                j = i * NBUF + b
                pltpu.make_async_copy(
                    h_hbm.at[src_v.at[j]], bufs.at[b], gsems.at[b]).wait()
                pltpu.async_copy(bufs.at[b], acc.at[dst_v.at[j]],
                                 ssems.at[b], add=True)
                if with_deg:
                    @pl.when(j > 0)
                    def _():
                        pltpu.make_async_copy(
                            ones_v, dacc.at[dst_v.at[j - 1]], dsem).wait()
                    pltpu.async_copy(ones_v, dacc.at[dst_v.at[j]], dsem,
                                     add=True)
                bg = (b - LAG) % NBUF
                g = j - LAG

                @pl.when((g >= 0) & (j < count - LAG))
                def _():
                    pltpu.make_async_copy(
                        bufs.at[bg], acc.at[dst_v.at[g]], ssems.at[bg]).wait()
                    gather_start(g + NBUF, bg)
            return _

        lax.fori_loop(0, count // NBUF, step, None)
        for b in range(NBUF):
            j = count - NBUF + b
            pltpu.make_async_copy(
                bufs.at[b], acc.at[dst_v.at[j]], ssems.at[b]).wait()
        if with_deg:
            pltpu.make_async_copy(
                ones_v, dacc.at[dst_v.at[count - 1]], dsem).wait()
        plsc.subcore_barrier()

        pltpu.sync_copy(acc.at[rows], out_hbm.at[c, rows])
        if with_deg:
            pltpu.sync_copy(dacc.at[rows], deg_out_hbm.at[c, rows])

    mesh = plsc.VectorSubcoreMesh(core_axis_name="c", subcore_axis_name="s",
                                  num_cores=NC, num_subcores=NS)
    return pl.kernel(body, out_type=tuple(out_type), mesh=mesh,
                     scratch_types=scratch,
                     compiler_params=pltpu.CompilerParams(
                         use_tc_tiling_on_sc=False))


_make_seg_pass = functools.lru_cache(maxsize=None)(_make_seg_pass)


def _row_blocks(nrows, width):
    return pl.BlockSpec((nrows, width), lambda i: (i, 0))


def _part_blocks(nrows, width):
    return pl.BlockSpec((NC, nrows, width), lambda i: (0, i, 0))


def _full(shape):
    return pl.BlockSpec(shape, lambda i: (0,) * len(shape))


_BLK = 1000
_GRID = N // _BLK


def _inv_deg(deg_ref):
    d = deg_ref[0, :, 0] + deg_ref[1, :, 0]
    return (1.0 / (d + 1.0))[:, None]


def _tc1_body(agg_ref, deg_ref, x_ref, w1_ref, b1_ref, w2_ref,
              g2_ref, g2b_ref):
    agg = agg_ref[0].astype(jnp.float32) + agg_ref[1].astype(jnp.float32)
    hn = (agg + x_ref[...]) * _inv_deg(deg_ref)
    h1 = jnp.maximum(
        jnp.dot(hn, w1_ref[...], preferred_element_type=jnp.float32)
        + b1_ref[...], 0.0)
    g2 = jnp.dot(h1, w2_ref[...], preferred_element_type=jnp.float32)
    g2_ref[...] = g2
    g2b_ref[...] = g2.astype(jnp.bfloat16)


def _tc2_body(q_ref, deg_ref, g2_ref, b2_ref, w3_ref, g3_ref):
    q = q_ref[0].astype(jnp.float32) + q_ref[1].astype(jnp.float32)
    h2 = jnp.maximum(
        (q + g2_ref[...]) * _inv_deg(deg_ref)
        + b2_ref[...], 0.0)
    g3_ref[...] = jnp.dot(h2, w3_ref[...], preferred_element_type=jnp.float32)


def _tc3_body(r_ref, deg_ref, g3_ref, b3_ref, wd1_ref, bd1_ref, wd2_ref,
              bd2_ref, wd3_ref, bd3_ref, xen_ref, xde_ref):
    xen = ((r_ref[0] + r_ref[1] + g3_ref[...]) * _inv_deg(deg_ref)
           + b3_ref[...])
    xen_ref[...] = xen
    d = jnp.maximum(
        jnp.dot(xen, wd1_ref[...], preferred_element_type=jnp.float32)
        + bd1_ref[...], 0.0)
    d = jnp.maximum(
        jnp.dot(d, wd2_ref[...], preferred_element_type=jnp.float32)
        + bd2_ref[...], 0.0)
    xde_ref[...] = (jnp.dot(d, wd3_ref[...], preferred_element_type=jnp.float32)
                    + bd3_ref[...])


def kernel(x, edge_index, W1, b1, W2, b2, W3, b3, Wd1, bd1, Wd2, bd2, Wd3, bd3):
    src = edge_index[0]
    dst = edge_index[1]
    pad = E_PAD - E
    srcp = jnp.concatenate([src, jnp.zeros((pad,), jnp.int32)]).reshape(
        E_PAD // K, K)
    dummy = N + jnp.arange(pad, dtype=jnp.int32) % (N_ACC - N)
    dstp = jnp.concatenate([dst, dummy]).reshape(E_PAD // K, K)
    zdeg = jnp.zeros((N_ACC, DEG_W), jnp.float32)
    ones = jnp.ones((K, DEG_W), jnp.float32)

    xb = x.astype(jnp.bfloat16)
    agg1, deg = _make_seg_pass(128, True, jnp.bfloat16, 124, 36)(
        xb, srcp, dstp, zdeg, ones)

    g2, g2b = pl.pallas_call(
        _tc1_body,
        grid=(_GRID,),
        in_specs=[_part_blocks(_BLK, 128), _part_blocks(_BLK, DEG_W),
                  _row_blocks(_BLK, 128), _full((128, 256)), _full((256,)),
                  _full((256, 64))],
        out_specs=[_row_blocks(_BLK, 64), _row_blocks(_BLK, 64)],
        out_shape=[jax.ShapeDtypeStruct((N, 64), jnp.float32),
                   jax.ShapeDtypeStruct((N, 64), jnp.bfloat16)],
    )(agg1, deg, x, W1, b1, W2)

    (agg2,) = _make_seg_pass(64, False, jnp.bfloat16, 96, 64)(
        g2b, srcp, dstp)

    g3 = pl.pallas_call(
        _tc2_body,
        grid=(_GRID,),
        in_specs=[_part_blocks(_BLK, 64), _part_blocks(_BLK, DEG_W),
                  _row_blocks(_BLK, 64), _full((64,)), _full((64, 16))],
        out_specs=_row_blocks(_BLK, 16),
        out_shape=jax.ShapeDtypeStruct((N, 16), jnp.float32),
    )(agg2, deg, g2, b2, W3)

    (agg3,) = _make_seg_pass(16, False, jnp.float32, 96, 64)(
        g3, srcp, dstp)

    x_en, x_de = pl.pallas_call(
        _tc3_body,
        grid=(_GRID,),
        in_specs=[_part_blocks(_BLK, 16), _part_blocks(_BLK, DEG_W),
                  _row_blocks(_BLK, 16), _full((16,)), _full((16, 64)),
                  _full((64,)), _full((64, 256)), _full((256,)),
                  _full((256, 128)), _full((128,))],
        out_specs=[_row_blocks(_BLK, 16), _row_blocks(_BLK, 128)],
        out_shape=[jax.ShapeDtypeStruct((N, 16), jnp.float32),
                   jax.ShapeDtypeStruct((N, 128), jnp.float32)],
    )(agg3, deg, g3, b3, Wd1, bd1, Wd2, bd2, Wd3, bd3)

    return (x_en, x_de)


# speedup vs baseline: 1.0960x; 1.0960x over previous
import functools

import jax
import jax.numpy as jnp
from jax import lax
from jax.experimental import pallas as pl
from jax.experimental.pallas import tpu as pltpu
from jax.experimental.pallas import tpu_sc as plsc

N = 10000
E = 320000
NC = 2
NS = 16
NW = NC * NS
K = 128
CPW = 80
E_PAD = NW * CPW * K
N_ACC = 10112
RPT = N_ACC // NS
DEG_W = 8
NBUF = 4
LAG = 2


def _make_seg_pass(D, with_deg, dtype, cpw0, cpw1):
    assert cpw0 % NBUF == 0 and cpw1 % NBUF == 0
    assert min(cpw0, cpw1) >= NBUF + LAG
    cpw_max = max(cpw0, cpw1)
    out_type = [jax.ShapeDtypeStruct((NC, N_ACC, D), dtype)]
    if with_deg:
        out_type.append(jax.ShapeDtypeStruct((NC, N_ACC, DEG_W), jnp.float32))
    scratch = [
        pltpu.VMEM((cpw_max, K), jnp.int32),
        pltpu.VMEM((cpw_max, K), jnp.int32),
        pltpu.VMEM((NBUF, K, D), dtype),
        pltpu.VMEM_SHARED((N_ACC, D), dtype),
        pltpu.SemaphoreType.DMA((NBUF,)),
        pltpu.SemaphoreType.DMA((NBUF,)),
    ]
    if with_deg:
        scratch.append(pltpu.VMEM((K, DEG_W), jnp.float32))
        scratch.append(pltpu.VMEM_SHARED((N_ACC, DEG_W), jnp.float32))
        scratch.append(pltpu.SemaphoreType.DMA)

    def body(h_hbm, srcp_hbm, dstp_hbm, zeros_hbm, *rest):
        if with_deg:
            (zeros_deg_hbm, ones_hbm, out_hbm, deg_out_hbm,
             src_v, dst_v, bufs, acc, gsems, ssems, ones_v, dacc,
             dsem) = rest
        else:
            (out_hbm, src_v, dst_v, bufs, acc, gsems, ssems) = rest
        c = lax.axis_index("c")
        s = lax.axis_index("s")
        count = jnp.where(c == 0, cpw0, cpw1)

        @pl.when(c == 0)
        def _():
            base = s * cpw0
            pltpu.sync_copy(srcp_hbm.at[pl.ds(base, cpw0)],
                            src_v.at[pl.ds(0, cpw0)])
            pltpu.sync_copy(dstp_hbm.at[pl.ds(base, cpw0)],
                            dst_v.at[pl.ds(0, cpw0)])

        @pl.when(c == 1)
        def _():
            base = NS * cpw0 + s * cpw1
            pltpu.sync_copy(srcp_hbm.at[pl.ds(base, cpw1)],
                            src_v.at[pl.ds(0, cpw1)])
            pltpu.sync_copy(dstp_hbm.at[pl.ds(base, cpw1)],
                            dst_v.at[pl.ds(0, cpw1)])

        rows = pl.ds(s * RPT, RPT)
        pltpu.sync_copy(zeros_hbm.at[rows], acc.at[rows])
        if with_deg:
            pltpu.sync_copy(zeros_deg_hbm.at[rows], dacc.at[rows])
            pltpu.sync_copy(ones_hbm, ones_v)
        plsc.subcore_barrier()

        def gather_start(j, b):
            pltpu.async_copy(h_hbm.at[src_v.at[j]], bufs.at[b], gsems.at[b])

        for b in range(NBUF):
            gather_start(b, b)

        def step(i, _):
            for b in range(NBUF):
                j = i * NBUF + b
                pltpu.make_async_copy(
                    h_hbm.at[src_v.at[j]], bufs.at[b], gsems.at[b]).wait()
                pltpu.async_copy(bufs.at[b], acc.at[dst_v.at[j]],
                                 ssems.at[b], add=True)
                if with_deg:
                    @pl.when(j > 0)
                    def _():
                        pltpu.make_async_copy(
                            ones_v, dacc.at[dst_v.at[j - 1]], dsem).wait()
                    pltpu.async_copy(ones_v, dacc.at[dst_v.at[j]], dsem,
                                     add=True)
                bg = (b - LAG) % NBUF
                g = j - LAG

                @pl.when((g >= 0) & (j < count - LAG))
                def _():
                    pltpu.make_async_copy(
                        bufs.at[bg], acc.at[dst_v.at[g]], ssems.at[bg]).wait()
                    gather_start(g + NBUF, bg)
            return _

        lax.fori_loop(0, count // NBUF, step, None)
        for b in range(NBUF):
            j = count - NBUF + b
            pltpu.make_async_copy(
                bufs.at[b], acc.at[dst_v.at[j]], ssems.at[b]).wait()
        if with_deg:
            pltpu.make_async_copy(
                ones_v, dacc.at[dst_v.at[count - 1]], dsem).wait()
        plsc.subcore_barrier()

        pltpu.sync_copy(acc.at[rows], out_hbm.at[c, rows])
        if with_deg:
            pltpu.sync_copy(dacc.at[rows], deg_out_hbm.at[c, rows])

    mesh = plsc.VectorSubcoreMesh(core_axis_name="c", subcore_axis_name="s",
                                  num_cores=NC, num_subcores=NS)
    return pl.kernel(body, out_type=tuple(out_type), mesh=mesh,
                     scratch_types=scratch,
                     compiler_params=pltpu.CompilerParams(
                         use_tc_tiling_on_sc=False))


_make_seg_pass = functools.lru_cache(maxsize=None)(_make_seg_pass)


def _row_blocks(nrows, width):
    return pl.BlockSpec((nrows, width), lambda i: (i, 0))


def _part_blocks(nrows, width):
    return pl.BlockSpec((NC, nrows, width), lambda i: (0, i, 0))


def _full(shape):
    return pl.BlockSpec(shape, lambda i: (0,) * len(shape))


_BLK = 1000
_GRID = N // _BLK


def _inv_deg(deg_ref):
    d = deg_ref[0, :, 0] + deg_ref[1, :, 0]
    return (1.0 / (d + 1.0))[:, None]


def _tc1_body(agg_ref, deg_ref, x_ref, w1_ref, b1_ref, w2_ref,
              g2_ref, g2b_ref):
    agg = agg_ref[0].astype(jnp.float32) + agg_ref[1].astype(jnp.float32)
    hn = (agg + x_ref[...]) * _inv_deg(deg_ref)
    h1 = jnp.maximum(
        jnp.dot(hn, w1_ref[...], preferred_element_type=jnp.float32)
        + b1_ref[...], 0.0)
    g2 = jnp.dot(h1, w2_ref[...], preferred_element_type=jnp.float32)
    g2_ref[...] = g2
    g2b_ref[...] = g2.astype(jnp.bfloat16)


def _tc2_body(q_ref, deg_ref, g2_ref, b2_ref, w3_ref, g3_ref):
    q = q_ref[0].astype(jnp.float32) + q_ref[1].astype(jnp.float32)
    h2 = jnp.maximum(
        (q + g2_ref[...]) * _inv_deg(deg_ref)
        + b2_ref[...], 0.0)
    g3_ref[...] = jnp.dot(h2, w3_ref[...], preferred_element_type=jnp.float32)


def _tc3_body(r_ref, deg_ref, g3_ref, b3_ref, wd1_ref, bd1_ref, wd2_ref,
              bd2_ref, wd3_ref, bd3_ref, xen_ref, xde_ref):
    xen = ((r_ref[0] + r_ref[1] + g3_ref[...]) * _inv_deg(deg_ref)
           + b3_ref[...])
    xen_ref[...] = xen
    d = jnp.maximum(
        jnp.dot(xen, wd1_ref[...], preferred_element_type=jnp.float32)
        + bd1_ref[...], 0.0)
    d = jnp.maximum(
        jnp.dot(d, wd2_ref[...], preferred_element_type=jnp.float32)
        + bd2_ref[...], 0.0)
    xde_ref[...] = (jnp.dot(d, wd3_ref[...], preferred_element_type=jnp.float32)
                    + bd3_ref[...])


def kernel(x, edge_index, W1, b1, W2, b2, W3, b3, Wd1, bd1, Wd2, bd2, Wd3, bd3):
    src = edge_index[0]
    dst = edge_index[1]
    pad = E_PAD - E
    srcp = jnp.concatenate([src, jnp.zeros((pad,), jnp.int32)]).reshape(
        E_PAD // K, K)
    dummy = N + jnp.arange(pad, dtype=jnp.int32) % (N_ACC - N)
    dstp = jnp.concatenate([dst, dummy]).reshape(E_PAD // K, K)
    z128b = jnp.zeros((N_ACC, 128), jnp.bfloat16)
    z64b = jnp.zeros((N_ACC, 64), jnp.bfloat16)
    z16 = jnp.zeros((N_ACC, 16), jnp.float32)
    zdeg = jnp.zeros((N_ACC, DEG_W), jnp.float32)
    ones = jnp.ones((K, DEG_W), jnp.float32)

    xb = x.astype(jnp.bfloat16)
    agg1, deg = _make_seg_pass(128, True, jnp.bfloat16, 124, 36)(
        xb, srcp, dstp, z128b, zdeg, ones)

    g2, g2b = pl.pallas_call(
        _tc1_body,
        grid=(_GRID,),
        in_specs=[_part_blocks(_BLK, 128), _part_blocks(_BLK, DEG_W),
                  _row_blocks(_BLK, 128), _full((128, 256)), _full((256,)),
                  _full((256, 64))],
        out_specs=[_row_blocks(_BLK, 64), _row_blocks(_BLK, 64)],
        out_shape=[jax.ShapeDtypeStruct((N, 64), jnp.float32),
                   jax.ShapeDtypeStruct((N, 64), jnp.bfloat16)],
    )(agg1, deg, x, W1, b1, W2)

    (agg2,) = _make_seg_pass(64, False, jnp.bfloat16, 112, 48)(
        g2b, srcp, dstp, z64b)

    g3 = pl.pallas_call(
        _tc2_body,
        grid=(_GRID,),
        in_specs=[_part_blocks(_BLK, 64), _part_blocks(_BLK, DEG_W),
                  _row_blocks(_BLK, 64), _full((64,)), _full((64, 16))],
        out_specs=_row_blocks(_BLK, 16),
        out_shape=jax.ShapeDtypeStruct((N, 16), jnp.float32),
    )(agg2, deg, g2, b2, W3)

    (agg3,) = _make_seg_pass(16, False, jnp.float32, 104, 56)(
        g3, srcp, dstp, z16)

    x_en, x_de = pl.pallas_call(
        _tc3_body,
        grid=(_GRID,),
        in_specs=[_part_blocks(_BLK, 16), _part_blocks(_BLK, DEG_W),
                  _row_blocks(_BLK, 16), _full((16,)), _full((16, 64)),
                  _full((64,)), _full((64, 256)), _full((256,)),
                  _full((256, 128)), _full((128,))],
        out_specs=[_row_blocks(_BLK, 16), _row_blocks(_BLK, 128)],
        out_shape=[jax.ShapeDtypeStruct((N, 16), jnp.float32),
                   jax.ShapeDtypeStruct((N, 128), jnp.float32)],
    )(agg3, deg, g3, b3, Wd1, bd1, Wd2, bd2, Wd3, bd3)

    return (x_en, x_de)
